# Initial kernel scaffold; baseline (speedup 1.0000x reference)
import jax
import jax.numpy as jnp
from jax.experimental import pallas as pl


def kernel(x, x_r, f, f_r, edge_index, edge_weight, msk, samp_bias1, samp_bias2, sparse, W_gcn, b_gcn, prelu_a, W_E, b_E, W_I, b_I, W_J, b_J):
    raise NotImplementedError("write your pallas kernel here")


# speedup vs baseline: 2.9555x; 2.9555x over previous
import functools

import jax
import jax.numpy as jnp
from jax import lax
from jax.experimental import pallas as pl
from jax.experimental.pallas import tpu as pltpu
from jax.experimental.pallas import tpu_sc as plsc

N = 10000
E = 320000
D = 128
NB = 10
BN = N // NB
NSUB = 16
EPT = E // NSUB
K = 80
NCHUNK = EPT // K
RPT = 624
ZR = 104

_f32 = jnp.float32


def _mm_body(x_ref, xr_ref, w_ref, o1_ref, o2_ref):
    w = w_ref[...]
    o1_ref[...] = jnp.dot(x_ref[...], w, preferred_element_type=_f32)
    o2_ref[...] = jnp.dot(xr_ref[...], w, preferred_element_type=_f32)


def _mm(x2, xr2, W):
    return pl.pallas_call(
        _mm_body,
        grid=(NB,),
        in_specs=[
            pl.BlockSpec((BN, D), lambda i: (i, 0)),
            pl.BlockSpec((BN, D), lambda i: (i, 0)),
            pl.BlockSpec((D, D), lambda i: (0, 0)),
        ],
        out_specs=[
            pl.BlockSpec((BN, D), lambda i: (i, 0)),
            pl.BlockSpec((BN, D), lambda i: (i, 0)),
        ],
        out_shape=[
            jax.ShapeDtypeStruct((N, D), _f32),
            jax.ShapeDtypeStruct((N, D), _f32),
        ],
    )(x2, xr2, W)


def _spmm_body(seq1, seq2, row2, col2, wrep3,
               out1, out2,
               accum, rowc, colc, wchunk, rows, zbuf, sem):
    c = lax.axis_index("c")
    s = lax.axis_index("s")

    def _zrow(i, carry):
        for q in range(D // 16):
            zbuf[i, pl.ds(q * 16, 16)] = jnp.zeros((16,), _f32)
        return carry
    lax.fori_loop(0, ZR, _zrow, 0)
    zbase = pl.multiple_of(s * RPT, 8)
    for p in range(RPT // ZR):
        pltpu.sync_copy(zbuf, accum.at[pl.ds(zbase + p * ZR, ZR)])

    @pl.when(s == NSUB - 1)
    def _():
        pltpu.sync_copy(zbuf.at[pl.ds(0, 16)],
                        accum.at[pl.ds(NSUB * RPT, 16)])

    plsc.subcore_barrier()

    def _edges(table):
        def _chunk(ci, carry):
            ch = s * NCHUNK + ci
            pltpu.sync_copy(row2.at[ch], rowc)
            pltpu.sync_copy(col2.at[ch], colc)
            pltpu.sync_copy(wrep3.at[ch], wchunk)
            pltpu.async_copy(table.at[colc], rows, sem).wait()

            def _edge(j, c2):
                wv = wchunk[j, pl.ds(0, 16)]
                for q in range(D // 16):
                    rows[j, pl.ds(q * 16, 16)] = (
                        rows[j, pl.ds(q * 16, 16)] * wv)
                return c2
            lax.fori_loop(0, K, _edge, 0)
            pltpu.sync_copy(rows, accum.at[rowc], add=True)
            return carry
        lax.fori_loop(0, NCHUNK, _chunk, 0)

    @pl.when(c == 0)
    def _():
        _edges(seq1)

    @pl.when(c == 1)
    def _():
        _edges(seq2)

    plsc.subcore_barrier()

    obase = pl.multiple_of(s * RPT, 8)

    @pl.when(c == 0)
    def _():
        pltpu.sync_copy(accum.at[pl.ds(obase, RPT)],
                        out1.at[pl.ds(obase, RPT)])

        @pl.when(s == NSUB - 1)
        def _():
            pltpu.sync_copy(accum.at[pl.ds(NSUB * RPT, 16)],
                            out1.at[pl.ds(NSUB * RPT, 16)])

    @pl.when(c == 1)
    def _():
        pltpu.sync_copy(accum.at[pl.ds(obase, RPT)],
                        out2.at[pl.ds(obase, RPT)])

        @pl.when(s == NSUB - 1)
        def _():
            pltpu.sync_copy(accum.at[pl.ds(NSUB * RPT, 16)],
                            out2.at[pl.ds(NSUB * RPT, 16)])


def _spmm(seq1, seq2, row, col, wrep):
    row2 = row.reshape(E // K, K)
    col2 = col.reshape(E // K, K)
    wrep3 = wrep.reshape(E // K, K, 16)
    mesh = plsc.VectorSubcoreMesh(core_axis_name="c", subcore_axis_name="s")
    fn = functools.partial(
        pl.kernel,
        mesh=mesh,
        out_type=[
            jax.ShapeDtypeStruct((N, D), _f32),
            jax.ShapeDtypeStruct((N, D), _f32),
        ],
        scratch_types=[
            pltpu.VMEM_SHARED((N, D), _f32),
            pltpu.VMEM((K,), jnp.int32),
            pltpu.VMEM((K,), jnp.int32),
            pltpu.VMEM((K, 16), _f32),
            pltpu.VMEM((K, D), _f32),
            pltpu.VMEM((ZR, D), _f32),
            pltpu.SemaphoreType.DMA,
        ],
    )(_spmm_body)
    return fn(seq1, seq2, row2, col2, wrep3)


def _c1_body(h1p, h2p, mskb, bg, a_ref, h1o, h2o, ssum, msum):
    i = pl.program_id(0)
    a = a_ref[0, 0]
    b = bg[...]
    h1 = h1p[...] + b
    h1 = jnp.where(h1 >= 0, h1, a * h1)
    h2 = h2p[...] + b
    h2 = jnp.where(h2 >= 0, h2, a * h2)
    h1o[...] = h1
    h2o[...] = h2
    m = mskb[...]

    @pl.when(i == 0)
    def _():
        ssum[...] = jnp.zeros_like(ssum)
        msum[...] = jnp.zeros_like(msum)

    ssum[...] += jnp.sum(h1 * m, axis=0, keepdims=True)
    msum[...] += jnp.sum(m).reshape(1, 1)


def _c1(h1p, h2p, mskc, bg2, a2):
    return pl.pallas_call(
        _c1_body,
        grid=(NB,),
        in_specs=[
            pl.BlockSpec((BN, D), lambda i: (i, 0)),
            pl.BlockSpec((BN, D), lambda i: (i, 0)),
            pl.BlockSpec((BN, 1), lambda i: (i, 0)),
            pl.BlockSpec((1, D), lambda i: (0, 0)),
            pl.BlockSpec((1, 1), lambda i: (0, 0)),
        ],
        out_specs=[
            pl.BlockSpec((BN, D), lambda i: (i, 0)),
            pl.BlockSpec((BN, D), lambda i: (i, 0)),
            pl.BlockSpec((1, D), lambda i: (0, 0)),
            pl.BlockSpec((1, 1), lambda i: (0, 0)),
        ],
        out_shape=[
            jax.ShapeDtypeStruct((N, D), _f32),
            jax.ShapeDtypeStruct((N, D), _f32),
            jax.ShapeDtypeStruct((1, D), _f32),
            jax.ShapeDtypeStruct((1, 1), _f32),
        ],
    )(h1p, h2p, mskc, bg2, a2)


def _c1b_body(ssum, msum, wE, s_o, vE_o):
    sv = jax.nn.sigmoid(ssum[...] / msum[0, 0])
    s_o[...] = sv
    vE_o[...] = jnp.sum(wE[...] * sv, axis=1)[None, :]


def _c1b(ssum, msum, W_E):
    return pl.pallas_call(
        _c1b_body,
        out_shape=[
            jax.ShapeDtypeStruct((1, D), _f32),
            jax.ShapeDtypeStruct((1, D), _f32),
        ],
    )(ssum, msum, W_E)


def _c2_body(h1, h2, fb, frb, s_ref, vE_ref, wI, wJ, sb1, sb2, bvec,
             e1, e2, i1, i2, j1, j2):
    sv = s_ref[...]
    vE = vE_ref[...]
    bE = bvec[0, 0]
    bI = bvec[0, 1]
    bJ = bvec[0, 2]
    h1v = h1[...]
    h2v = h2[...]
    fv = fb[...]
    frv = frb[...]
    s1 = sb1[...]
    s2 = sb2[...]
    e1[...] = jnp.sum(h1v * vE, axis=1, keepdims=True) + bE + s1
    e2[...] = jnp.sum(h2v * vE, axis=1, keepdims=True) + bE + s2
    P = jnp.dot(h1v, wI[...], preferred_element_type=_f32)
    i1[...] = jnp.sum(P * fv, axis=1, keepdims=True) + bI + s1
    i2[...] = jnp.sum(P * frv, axis=1, keepdims=True) + bI + s2
    Q = jnp.dot(h1v * sv, wJ[...], preferred_element_type=_f32)
    j1[...] = jnp.sum(Q * fv, axis=1, keepdims=True) + bJ + s1
    j2[...] = jnp.sum(Q * frv, axis=1, keepdims=True) + bJ + s2


def _c2(h1, h2, f2, fr2, s, vE, W_I, W_J, sb1, sb2, bvec):
    vec = lambda: pl.BlockSpec((BN, 1), lambda i: (i, 0))
    blk = lambda: pl.BlockSpec((BN, D), lambda i: (i, 0))
    fix = lambda r, c: pl.BlockSpec((r, c), lambda i: (0, 0))
    return pl.pallas_call(
        _c2_body,
        grid=(NB,),
        in_specs=[
            blk(), blk(), blk(), blk(),
            fix(1, D), fix(1, D), fix(D, D), fix(D, D),
            vec(), vec(), fix(1, 3),
        ],
        out_specs=[vec() for _ in range(6)],
        out_shape=[jax.ShapeDtypeStruct((N, 1), _f32) for _ in range(6)],
    )(h1, h2, f2, fr2, s, vE, W_I, W_J, sb1, sb2, bvec)


def kernel(x, x_r, f, f_r, edge_index, edge_weight, msk, samp_bias1,
           samp_bias2, sparse, W_gcn, b_gcn, prelu_a, W_E, b_E, W_I, b_I,
           W_J, b_J):
    x2 = x[0]
    xr2 = x_r[0]
    f2 = f[0]
    fr2 = f_r[0]
    row = edge_index[0]
    col = edge_index[1]

    seq1, seq2 = _mm(x2, xr2, W_gcn)
    wrep = jnp.broadcast_to(edge_weight[:, None], (E, 16))
    h1p, h2p = _spmm(seq1, seq2, row, col, wrep)

    bg2 = b_gcn.reshape(1, D)
    a2 = prelu_a.reshape(1, 1)
    mskc = msk.reshape(N, 1)
    h1, h2, ssum, msum = _c1(h1p, h2p, mskc, bg2, a2)
    s, vE = _c1b(ssum, msum, W_E)

    bvec = jnp.stack([b_E, b_I, b_J]).reshape(1, 3)
    e1, e2, i1, i2, j1, j2 = _c2(h1, h2, f2, fr2, s, vE, W_I, W_J,
                                 samp_bias1.reshape(N, 1),
                                 samp_bias2.reshape(N, 1), bvec)

    ret_E = jnp.concatenate([e1, e2]).reshape(1, 2 * N)
    ret_I = jnp.concatenate([i1, i2]).reshape(1, 2 * N)
    ret_J = jnp.concatenate([j1, j2]).reshape(1, 2 * N)
    return (ret_E, ret_I, ret_J)


# speedup vs baseline: 5.2417x; 1.7735x over previous
import functools

import jax
import jax.numpy as jnp
from jax import lax
from jax.experimental import pallas as pl
from jax.experimental.pallas import tpu as pltpu
from jax.experimental.pallas import tpu_sc as plsc

N = 10000
E = 320000
D = 128
NB = 10
BN = N // NB
NSUB = 16
EPT = E // NSUB
K = 40
NCHUNK = EPT // K
RPT = 624
ZR = 104

_f32 = jnp.float32


def _mm_body(x_ref, xr_ref, w_ref, o1_ref, o2_ref):
    w = w_ref[...]
    o1_ref[...] = jnp.dot(x_ref[...], w, preferred_element_type=_f32)
    o2_ref[...] = jnp.dot(xr_ref[...], w, preferred_element_type=_f32)


def _mm(x2, xr2, W):
    return pl.pallas_call(
        _mm_body,
        grid=(NB,),
        in_specs=[
            pl.BlockSpec((BN, D), lambda i: (i, 0)),
            pl.BlockSpec((BN, D), lambda i: (i, 0)),
            pl.BlockSpec((D, D), lambda i: (0, 0)),
        ],
        out_specs=[
            pl.BlockSpec((BN, D), lambda i: (i, 0)),
            pl.BlockSpec((BN, D), lambda i: (i, 0)),
        ],
        out_shape=[
            jax.ShapeDtypeStruct((N, D), _f32),
            jax.ShapeDtypeStruct((N, D), _f32),
        ],
    )(x2, xr2, W)


def _spmm_body(seq1, seq2, row2, col2, wrep3,
               out1, out2,
               accum,
               rowc0, colc0, wch0, rowc1, colc1, wch1,
               rowc2, colc2, wch2, rowc3, colc3, wch3,
               rows0, rows1, zbuf,
               isem0, isem1, isem2, isem3,
               gsem0, gsem1, ssem0, ssem1):
    c = lax.axis_index("c")
    s = lax.axis_index("s")
    idxb = ((rowc0, colc0, wch0, isem0),
            (rowc1, colc1, wch1, isem1),
            (rowc2, colc2, wch2, isem2),
            (rowc3, colc3, wch3, isem3))
    rowb = ((rows0, gsem0, ssem0),
            (rows1, gsem1, ssem1))

    def _zrow(i, carry):
        for q in range(D // 16):
            zbuf[i, pl.ds(q * 16, 16)] = jnp.zeros((16,), _f32)
        return carry
    lax.fori_loop(0, ZR, _zrow, 0)
    zbase = pl.multiple_of(s * RPT, 8)
    for p in range(RPT // ZR):
        pltpu.sync_copy(zbuf, accum.at[pl.ds(zbase + p * ZR, ZR)])

    @pl.when(s == NSUB - 1)
    def _():
        pltpu.sync_copy(zbuf.at[pl.ds(0, 16)],
                        accum.at[pl.ds(NSUB * RPT, 16)])

    plsc.subcore_barrier()

    base = s * NCHUNK

    def _issue_idx(k, ch):
        rowc, colc, wch, isem = idxb[k]
        pltpu.async_copy(row2.at[ch], rowc, isem)
        pltpu.async_copy(col2.at[ch], colc, isem)
        pltpu.async_copy(wrep3.at[ch], wch, isem)

    def _wait_idx(k, ch):
        rowc, colc, wch, isem = idxb[k]
        pltpu.make_async_copy(row2.at[ch], rowc, isem).wait()
        pltpu.make_async_copy(col2.at[ch], colc, isem).wait()
        pltpu.make_async_copy(wrep3.at[ch], wch, isem).wait()

    def _edges(table):
        def _issue_gather(m, k):
            rows, gsem, _s = rowb[m]
            colc = idxb[k][1]
            pltpu.async_copy(table.at[colc], rows, gsem)

        def _wait_gather(m, k):
            rows, gsem, _s = rowb[m]
            colc = idxb[k][1]
            pltpu.make_async_copy(table.at[colc], rows, gsem).wait()

        def _issue_scatter(m, k):
            rows, _g, ssem = rowb[m]
            rowc = idxb[k][0]
            pltpu.async_copy(rows, accum.at[rowc], ssem, add=True)

        def _wait_scatter(m, k):
            rows, _g, ssem = rowb[m]
            rowc = idxb[k][0]
            pltpu.make_async_copy(rows, accum.at[rowc], ssem).wait()

        def _scale(m, k):
            rows = rowb[m][0]
            wch = idxb[k][2]

            def _edge(j, c2):
                wv = wch[j, pl.ds(0, 16)]
                for q in range(D // 16):
                    rows[j, pl.ds(q * 16, 16)] = (
                        rows[j, pl.ds(q * 16, 16)] * wv)
                return c2
            lax.fori_loop(0, K, _edge, 0)

        _issue_idx(0, base)
        _wait_idx(0, base)
        _issue_gather(0, 0)
        _issue_idx(1, base + 1)

        def _quad(g, carry):
            for off in range(4):
                m, mn = off % 2, (off + 1) % 2
                k, kn, ki = off, (off + 1) % 4, (off + 2) % 4
                ch = base + 4 * g + off
                _wait_idx(kn, ch + 1)
                if off == 0:
                    @pl.when(g >= 1)
                    def _():
                        _wait_scatter(mn, (off + 3) % 4)
                else:
                    _wait_scatter(mn, (off + 3) % 4)
                _issue_idx(ki, ch + 2)
                _issue_gather(mn, kn)
                _wait_gather(m, k)
                _scale(m, k)
                _issue_scatter(m, k)
            return carry
        lax.fori_loop(0, NCHUNK // 4 - 1, _quad, 0)

        cb = base + NCHUNK - 4
        _wait_idx(1, cb + 1)
        _wait_scatter(1, 3)
        _issue_idx(2, cb + 2)
        _issue_gather(1, 1)
        _wait_gather(0, 0)
        _scale(0, 0)
        _issue_scatter(0, 0)

        _wait_idx(2, cb + 2)
        _wait_scatter(0, 0)
        _issue_idx(3, cb + 3)
        _issue_gather(0, 2)
        _wait_gather(1, 1)
        _scale(1, 1)
        _issue_scatter(1, 1)

        _wait_idx(3, cb + 3)
        _wait_scatter(1, 1)
        _issue_gather(1, 3)
        _wait_gather(0, 2)
        _scale(0, 2)
        _issue_scatter(0, 2)

        _wait_scatter(0, 2)
        _wait_gather(1, 3)
        _scale(1, 3)
        _issue_scatter(1, 3)
        _wait_scatter(1, 3)

    @pl.when(c == 0)
    def _():
        _edges(seq1)

    @pl.when(c == 1)
    def _():
        _edges(seq2)

    plsc.subcore_barrier()

    obase = pl.multiple_of(s * RPT, 8)

    @pl.when(c == 0)
    def _():
        pltpu.sync_copy(accum.at[pl.ds(obase, RPT)],
                        out1.at[pl.ds(obase, RPT)])

        @pl.when(s == NSUB - 1)
        def _():
            pltpu.sync_copy(accum.at[pl.ds(NSUB * RPT, 16)],
                            out1.at[pl.ds(NSUB * RPT, 16)])

    @pl.when(c == 1)
    def _():
        pltpu.sync_copy(accum.at[pl.ds(obase, RPT)],
                        out2.at[pl.ds(obase, RPT)])

        @pl.when(s == NSUB - 1)
        def _():
            pltpu.sync_copy(accum.at[pl.ds(NSUB * RPT, 16)],
                            out2.at[pl.ds(NSUB * RPT, 16)])


def _spmm(seq1, seq2, row, col, wrep):
    row2 = row.reshape(E // K, K)
    col2 = col.reshape(E // K, K)
    wrep3 = wrep.reshape(E // K, K, 16)
    mesh = plsc.VectorSubcoreMesh(core_axis_name="c", subcore_axis_name="s")
    fn = functools.partial(
        pl.kernel,
        mesh=mesh,
        out_type=[
            jax.ShapeDtypeStruct((N, D), _f32),
            jax.ShapeDtypeStruct((N, D), _f32),
        ],
        scratch_types=(
            [pltpu.VMEM_SHARED((N, D), _f32)]
            + [pltpu.VMEM((K,), jnp.int32),
               pltpu.VMEM((K,), jnp.int32),
               pltpu.VMEM((K, 16), _f32)] * 4
            + [pltpu.VMEM((K, D), _f32),
               pltpu.VMEM((K, D), _f32),
               pltpu.VMEM((ZR, D), _f32)]
            + [pltpu.SemaphoreType.DMA] * 8
        ),
    )(_spmm_body)
    return fn(seq1, seq2, row2, col2, wrep3)


def _c1_body(h1p, h2p, mskb, bg, a_ref, h1o, h2o, ssum, msum):
    i = pl.program_id(0)
    a = a_ref[0, 0]
    b = bg[...]
    h1 = h1p[...] + b
    h1 = jnp.where(h1 >= 0, h1, a * h1)
    h2 = h2p[...] + b
    h2 = jnp.where(h2 >= 0, h2, a * h2)
    h1o[...] = h1
    h2o[...] = h2
    m = mskb[...]

    @pl.when(i == 0)
    def _():
        ssum[...] = jnp.zeros_like(ssum)
        msum[...] = jnp.zeros_like(msum)

    ssum[...] += jnp.sum(h1 * m, axis=0, keepdims=True)
    msum[...] += jnp.sum(m).reshape(1, 1)


def _c1(h1p, h2p, mskc, bg2, a2):
    return pl.pallas_call(
        _c1_body,
        grid=(NB,),
        in_specs=[
            pl.BlockSpec((BN, D), lambda i: (i, 0)),
            pl.BlockSpec((BN, D), lambda i: (i, 0)),
            pl.BlockSpec((BN, 1), lambda i: (i, 0)),
            pl.BlockSpec((1, D), lambda i: (0, 0)),
            pl.BlockSpec((1, 1), lambda i: (0, 0)),
        ],
        out_specs=[
            pl.BlockSpec((BN, D), lambda i: (i, 0)),
            pl.BlockSpec((BN, D), lambda i: (i, 0)),
            pl.BlockSpec((1, D), lambda i: (0, 0)),
            pl.BlockSpec((1, 1), lambda i: (0, 0)),
        ],
        out_shape=[
            jax.ShapeDtypeStruct((N, D), _f32),
            jax.ShapeDtypeStruct((N, D), _f32),
            jax.ShapeDtypeStruct((1, D), _f32),
            jax.ShapeDtypeStruct((1, 1), _f32),
        ],
    )(h1p, h2p, mskc, bg2, a2)


def _c1b_body(ssum, msum, wE, s_o, vE_o):
    sv = jax.nn.sigmoid(ssum[...] / msum[0, 0])
    s_o[...] = sv
    vE_o[...] = jnp.sum(wE[...] * sv, axis=1)[None, :]


def _c1b(ssum, msum, W_E):
    return pl.pallas_call(
        _c1b_body,
        out_shape=[
            jax.ShapeDtypeStruct((1, D), _f32),
            jax.ShapeDtypeStruct((1, D), _f32),
        ],
    )(ssum, msum, W_E)


def _c2_body(h1, h2, fb, frb, s_ref, vE_ref, wI, wJ, sb1, sb2, bvec,
             e1, e2, i1, i2, j1, j2):
    sv = s_ref[...]
    vE = vE_ref[...]
    bE = bvec[0, 0]
    bI = bvec[0, 1]
    bJ = bvec[0, 2]
    h1v = h1[...]
    h2v = h2[...]
    fv = fb[...]
    frv = frb[...]
    s1 = sb1[...]
    s2 = sb2[...]
    e1[...] = jnp.sum(h1v * vE, axis=1, keepdims=True) + bE + s1
    e2[...] = jnp.sum(h2v * vE, axis=1, keepdims=True) + bE + s2
    P = jnp.dot(h1v, wI[...], preferred_element_type=_f32)
    i1[...] = jnp.sum(P * fv, axis=1, keepdims=True) + bI + s1
    i2[...] = jnp.sum(P * frv, axis=1, keepdims=True) + bI + s2
    Q = jnp.dot(h1v * sv, wJ[...], preferred_element_type=_f32)
    j1[...] = jnp.sum(Q * fv, axis=1, keepdims=True) + bJ + s1
    j2[...] = jnp.sum(Q * frv, axis=1, keepdims=True) + bJ + s2


def _c2(h1, h2, f2, fr2, s, vE, W_I, W_J, sb1, sb2, bvec):
    vec = lambda: pl.BlockSpec((BN, 1), lambda i: (i, 0))
    blk = lambda: pl.BlockSpec((BN, D), lambda i: (i, 0))
    fix = lambda r, c: pl.BlockSpec((r, c), lambda i: (0, 0))
    return pl.pallas_call(
        _c2_body,
        grid=(NB,),
        in_specs=[
            blk(), blk(), blk(), blk(),
            fix(1, D), fix(1, D), fix(D, D), fix(D, D),
            vec(), vec(), fix(1, 3),
        ],
        out_specs=[vec() for _ in range(6)],
        out_shape=[jax.ShapeDtypeStruct((N, 1), _f32) for _ in range(6)],
    )(h1, h2, f2, fr2, s, vE, W_I, W_J, sb1, sb2, bvec)


def kernel(x, x_r, f, f_r, edge_index, edge_weight, msk, samp_bias1,
           samp_bias2, sparse, W_gcn, b_gcn, prelu_a, W_E, b_E, W_I, b_I,
           W_J, b_J):
    x2 = x[0]
    xr2 = x_r[0]
    f2 = f[0]
    fr2 = f_r[0]
    row = edge_index[0]
    col = edge_index[1]

    seq1, seq2 = _mm(x2, xr2, W_gcn)
    wrep = jnp.broadcast_to(edge_weight[:, None], (E, 16))
    h1p, h2p = _spmm(seq1, seq2, row, col, wrep)

    bg2 = b_gcn.reshape(1, D)
    a2 = prelu_a.reshape(1, 1)
    mskc = msk.reshape(N, 1)
    h1, h2, ssum, msum = _c1(h1p, h2p, mskc, bg2, a2)
    s, vE = _c1b(ssum, msum, W_E)

    bvec = jnp.stack([b_E, b_I, b_J]).reshape(1, 3)
    e1, e2, i1, i2, j1, j2 = _c2(h1, h2, f2, fr2, s, vE, W_I, W_J,
                                 samp_bias1.reshape(N, 1),
                                 samp_bias2.reshape(N, 1), bvec)

    ret_E = jnp.concatenate([e1, e2]).reshape(1, 2 * N)
    ret_I = jnp.concatenate([i1, i2]).reshape(1, 2 * N)
    ret_J = jnp.concatenate([j1, j2]).reshape(1, 2 * N)
    return (ret_E, ret_I, ret_J)
